# Initial kernel scaffold; baseline (speedup 1.0000x reference)
import functools

import jax
import jax.numpy as jnp
from jax.experimental import pallas as pl

E = 8
TOP_K = 2
D = 1024
I = 512
S = 2048
T = 2048

BT = 256


def _fused_kernel(x_ref, rw_ref, egu_ref, ed_ref, sg_ref, su_ref, sd_ref, seg_ref,
                  out_ref):
    xb = x_ref[...]
    xb_bf = xb.astype(jnp.bfloat16)

    logits = jax.lax.dot_general(
        xb, rw_ref[...], (((1,), (1,)), ((), ())),
        preferred_element_type=jnp.float32,
        precision=jax.lax.Precision.HIGHEST)
    m = jnp.max(logits, axis=-1, keepdims=True)
    ex = jnp.exp(logits - m)
    probs = ex / jnp.sum(ex, axis=-1, keepdims=True)

    eids = jax.lax.broadcasted_iota(jnp.int32, (BT, E), 1)
    i1 = jnp.argmax(probs, axis=-1)[:, None]
    oh1 = (eids == i1)
    v1 = jnp.max(probs, axis=-1, keepdims=True)
    masked = jnp.where(oh1, -jnp.inf, probs)
    i2 = jnp.argmax(masked, axis=-1)[:, None]
    oh2 = (eids == i2)
    v2 = jnp.max(masked, axis=-1, keepdims=True)
    denom = v1 + v2
    combine = (v1 / denom) * oh1.astype(jnp.float32) + \
              (v2 / denom) * oh2.astype(jnp.float32)

    acc = jnp.zeros((BT, D), dtype=jnp.float32)
    for e in range(E):
        gu = jax.lax.dot_general(
            xb_bf, egu_ref[e], (((1,), (1,)), ((), ())),
            preferred_element_type=jnp.float32)
        h = jax.nn.silu(gu[:, :I]) * gu[:, I:]
        eo = jax.lax.dot_general(
            h.astype(jnp.bfloat16), ed_ref[e], (((1,), (1,)), ((), ())),
            preferred_element_type=jnp.float32)
        acc = acc + combine[:, e:e + 1] * eo

    sg = jax.lax.dot_general(
        xb_bf, sg_ref[...], (((1,), (0,)), ((), ())),
        preferred_element_type=jnp.float32)
    su = jax.lax.dot_general(
        xb_bf, su_ref[...], (((1,), (0,)), ((), ())),
        preferred_element_type=jnp.float32)
    sh = jax.nn.silu(sg) * su
    so = jax.lax.dot_general(
        sh.astype(jnp.bfloat16), sd_ref[...], (((1,), (0,)), ((), ())),
        preferred_element_type=jnp.float32)
    glogit = jax.lax.dot_general(
        xb, seg_ref[...], (((1,), (1,)), ((), ())),
        preferred_element_type=jnp.float32)
    g = jax.nn.sigmoid(glogit)
    out_ref[...] = acc + g * so


def kernel(x, router_weight, expert_gate_up, expert_down, shared_gate,
           shared_up, shared_down, shared_expert_gate):
    egu = expert_gate_up.astype(jnp.bfloat16)
    ed = expert_down.astype(jnp.bfloat16)
    sg = shared_gate.astype(jnp.bfloat16)
    su = shared_up.astype(jnp.bfloat16)
    sd = shared_down.astype(jnp.bfloat16)
    seg = shared_expert_gate.reshape(1, D)

    grid = (T // BT,)
    out = pl.pallas_call(
        _fused_kernel,
        grid=grid,
        in_specs=[
            pl.BlockSpec((BT, D), lambda i: (i, 0)),
            pl.BlockSpec((E, D), lambda i: (0, 0)),
            pl.BlockSpec((E, 2 * I, D), lambda i: (0, 0, 0)),
            pl.BlockSpec((E, D, I), lambda i: (0, 0, 0)),
            pl.BlockSpec((D, S), lambda i: (0, 0)),
            pl.BlockSpec((D, S), lambda i: (0, 0)),
            pl.BlockSpec((S, D), lambda i: (0, 0)),
            pl.BlockSpec((1, D), lambda i: (0, 0)),
        ],
        out_specs=pl.BlockSpec((BT, D), lambda i: (i, 0)),
        out_shape=jax.ShapeDtypeStruct((T, D), jnp.float32),
    )(x, router_weight, egu, ed, sg, su, sd, seg)
    return out


# speedup vs baseline: 1.7115x; 1.7115x over previous
import functools

import jax
import jax.numpy as jnp
from jax.experimental import pallas as pl

E = 8
TOP_K = 2
D = 1024
I = 512
S = 2048
T = 2048

BT = 256


def _fused_kernel(x_ref, rw_ref, egu_ref, ed_ref, sg_ref, su_ref, sd_ref, seg_ref,
                  out_ref):
    xb = x_ref[...]
    xb_bf = xb.astype(jnp.bfloat16)

    logits = jax.lax.dot_general(
        xb_bf, rw_ref[...].astype(jnp.bfloat16), (((1,), (1,)), ((), ())),
        preferred_element_type=jnp.float32)
    m = jnp.max(logits, axis=-1, keepdims=True)
    ex = jnp.exp(logits - m)
    probs = ex / jnp.sum(ex, axis=-1, keepdims=True)

    eids = jax.lax.broadcasted_iota(jnp.int32, (BT, E), 1)
    i1 = jnp.argmax(probs, axis=-1)[:, None]
    oh1 = (eids == i1)
    v1 = jnp.max(probs, axis=-1, keepdims=True)
    masked = jnp.where(oh1, -jnp.inf, probs)
    i2 = jnp.argmax(masked, axis=-1)[:, None]
    oh2 = (eids == i2)
    v2 = jnp.max(masked, axis=-1, keepdims=True)
    denom = v1 + v2
    combine = (v1 / denom) * oh1.astype(jnp.float32) + \
              (v2 / denom) * oh2.astype(jnp.float32)

    acc = jnp.zeros((BT, D), dtype=jnp.float32)
    for e in range(E):
        gu = jax.lax.dot_general(
            xb_bf, egu_ref[e], (((1,), (1,)), ((), ())),
            preferred_element_type=jnp.float32)
        h = jax.nn.silu(gu[:, :I]) * gu[:, I:]
        eo = jax.lax.dot_general(
            h.astype(jnp.bfloat16), ed_ref[e], (((1,), (1,)), ((), ())),
            preferred_element_type=jnp.float32)
        acc = acc + combine[:, e:e + 1] * eo

    sg = jax.lax.dot_general(
        xb_bf, sg_ref[...], (((1,), (0,)), ((), ())),
        preferred_element_type=jnp.float32)
    su = jax.lax.dot_general(
        xb_bf, su_ref[...], (((1,), (0,)), ((), ())),
        preferred_element_type=jnp.float32)
    sh = jax.nn.silu(sg) * su
    so = jax.lax.dot_general(
        sh.astype(jnp.bfloat16), sd_ref[...], (((1,), (0,)), ((), ())),
        preferred_element_type=jnp.float32)
    glogit = jax.lax.dot_general(
        xb, seg_ref[...], (((1,), (1,)), ((), ())),
        preferred_element_type=jnp.float32)
    g = jax.nn.sigmoid(glogit)
    out_ref[...] = acc + g * so


def kernel(x, router_weight, expert_gate_up, expert_down, shared_gate,
           shared_up, shared_down, shared_expert_gate):
    egu = expert_gate_up.astype(jnp.bfloat16)
    ed = expert_down.astype(jnp.bfloat16)
    sg = shared_gate.astype(jnp.bfloat16)
    su = shared_up.astype(jnp.bfloat16)
    sd = shared_down.astype(jnp.bfloat16)
    seg = shared_expert_gate.reshape(1, D)

    grid = (T // BT,)
    out = pl.pallas_call(
        _fused_kernel,
        grid=grid,
        in_specs=[
            pl.BlockSpec((BT, D), lambda i: (i, 0)),
            pl.BlockSpec((E, D), lambda i: (0, 0)),
            pl.BlockSpec((E, 2 * I, D), lambda i: (0, 0, 0)),
            pl.BlockSpec((E, D, I), lambda i: (0, 0, 0)),
            pl.BlockSpec((D, S), lambda i: (0, 0)),
            pl.BlockSpec((D, S), lambda i: (0, 0)),
            pl.BlockSpec((S, D), lambda i: (0, 0)),
            pl.BlockSpec((1, D), lambda i: (0, 0)),
        ],
        out_specs=pl.BlockSpec((BT, D), lambda i: (i, 0)),
        out_shape=jax.ShapeDtypeStruct((T, D), jnp.float32),
    )(x, router_weight, egu, ed, sg, su, sd, seg)
    return out
